# baseline (speedup 1.0000x reference)
import functools

import jax
import jax.numpy as jnp
from jax import lax
from jax.experimental import pallas as pl
from jax.experimental.pallas import tpu as pltpu
from jax.experimental.pallas import tpu_sc as plsc

N = 10000
D = 256
E = 160000
C = 128
NSUB = 16
NCH_T = 80
EPAD = NSUB * NCH_T * C
ZR = 200
NZCH = N // ZR


def _spmm2_sc(ha_lo, ha_hi, hb_lo, hb_hi, ea, eb):
  mesh = plsc.VectorSubcoreMesh(core_axis_name="c", subcore_axis_name="s")

  @functools.partial(
      pl.kernel,
      out_type=[jax.ShapeDtypeStruct((N, 128), jnp.float32)] * 4,
      mesh=mesh,
      scratch_types=[
          pltpu.VMEM((NCH_T // 2, C), jnp.int32),
          pltpu.VMEM((NCH_T // 2, C), jnp.int32),
          pltpu.VMEM((NCH_T // 2, C), jnp.float32),
          pltpu.VMEM((C, 128), jnp.float32),
          pltpu.VMEM((C, 128), jnp.float32),
          pltpu.VMEM_SHARED((N, 128), jnp.float32),
          pltpu.SemaphoreType.DMA,
          pltpu.SemaphoreType.DMA,
      ],
  )
  def k(halo, hahi, hblo, hbhi, rwsa, clsa, wsa, rwsb, clsb, wsb,
        oalo, oahi, oblo, obhi,
        colr, rowr, wr, g0, g1, acc, gsem0, gsem1):
    c = lax.axis_index("c")
    s = lax.axis_index("s")
    hc = NCH_T // 2

    def zero_acc():
      @pl.loop(0, 80)
      def _(i):
        for kk in range(8):
          g0[i, pl.ds(kk * 16, 16)] = jnp.zeros((16,), jnp.float32)

      zsrc = g0.at[pl.ds(0, 80)]

      @pl.loop(s, 125, step=NSUB)
      def _(j):
        pltpu.sync_copy(zsrc, acc.at[pl.ds(j * 80, 80)])

    def run(h_ref, rws, cls, ws):
      bufs = ((g0, gsem0), (g1, gsem1))
      for half in range(2):
        tsl = pl.ds(s * NCH_T + half * hc, hc)
        pltpu.sync_copy(cls.at[tsl], colr)
        pltpu.sync_copy(rws.at[tsl], rowr)
        pltpu.sync_copy(ws.at[tsl], wr)

        pltpu.async_copy(h_ref.at[colr.at[0]], g0, gsem0)
        pltpu.async_copy(h_ref.at[colr.at[1]], g1, gsem1)

        @pl.loop(0, hc // 2)
        def _(i):
          for p, (gbuf, gsem) in enumerate(bufs):
            ck = 2 * i + p
            pltpu.make_async_copy(h_ref.at[colr.at[ck]], gbuf, gsem).wait()

            @pl.loop(0, C // 16)
            def _(b):
              wvec = wr[ck, pl.ds(b * 16, 16)]
              for ii in range(16):
                w = wvec[ii]
                for kk in range(8):
                  sl = pl.ds(kk * 16, 16)
                  gbuf[b * 16 + ii, sl] = gbuf[b * 16 + ii, sl] * w

            pltpu.sync_copy(gbuf, acc.at[rowr.at[ck]], add=True)

            @pl.when(ck + 2 < hc)
            def _():
              pltpu.async_copy(h_ref.at[colr.at[ck + 2]], gbuf, gsem)

    def writeout(olo, ohi):
      @pl.loop(s, NZCH, step=NSUB)
      def _(j):
        sl = pl.ds(j * ZR, ZR)

        @pl.when(c == 0)
        def _():
          pltpu.sync_copy(acc.at[sl], olo.at[sl])

        @pl.when(c == 1)
        def _():
          pltpu.sync_copy(acc.at[sl], ohi.at[sl])

    for (hlo, hhi, rws, cls, ws, olo, ohi) in (
        (halo, hahi, rwsa, clsa, wsa, oalo, oahi),
        (hblo, hbhi, rwsb, clsb, wsb, oblo, obhi)):
      zero_acc()
      plsc.subcore_barrier()

      @pl.when(c == 0)
      def _():
        run(hlo, rws, cls, ws)

      @pl.when(c == 1)
      def _():
        run(hhi, rws, cls, ws)

      plsc.subcore_barrier()
      writeout(olo, ohi)
      plsc.subcore_barrier()

  return k(ha_lo, ha_hi, hb_lo, hb_hi, *ea, *eb)


_BM = 1000


def _lin_relu_mask(s0, s1, wt, mask):
  def body(a0, a1, w, m, o0, o1):
    acc = jnp.dot(a0[...], w[0], preferred_element_type=jnp.float32)
    acc = acc + jnp.dot(a1[...], w[1], preferred_element_type=jnp.float32)
    y = jnp.maximum(acc, 0.0) * m[...]
    o0[...] = y[:, :128]
    o1[...] = y[:, 128:]

  return pl.pallas_call(
      body,
      grid=(N // _BM,),
      in_specs=[
          pl.BlockSpec((_BM, 128), lambda i: (i, 0)),
          pl.BlockSpec((_BM, 128), lambda i: (i, 0)),
          pl.BlockSpec((2, 128, D), lambda i: (0, 0, 0)),
          pl.BlockSpec((_BM, D), lambda i: (i, 0)),
      ],
      out_specs=[
          pl.BlockSpec((_BM, 128), lambda i: (i, 0)),
          pl.BlockSpec((_BM, 128), lambda i: (i, 0)),
      ],
      out_shape=[jax.ShapeDtypeStruct((N, 128), jnp.float32)] * 2,
  )(s0, s1, wt, mask)


def _lin_plain(s0, s1, wt):
  ow = wt.shape[2]

  def body(a0, a1, w, o):
    acc = jnp.dot(a0[...], w[0], preferred_element_type=jnp.float32)
    acc = acc + jnp.dot(a1[...], w[1], preferred_element_type=jnp.float32)
    o[...] = acc

  return pl.pallas_call(
      body,
      grid=(N // _BM,),
      in_specs=[
          pl.BlockSpec((_BM, 128), lambda i: (i, 0)),
          pl.BlockSpec((_BM, 128), lambda i: (i, 0)),
          pl.BlockSpec((2, 128, ow), lambda i: (0, 0, 0)),
      ],
      out_specs=pl.BlockSpec((_BM, ow), lambda i: (i, 0)),
      out_shape=jax.ShapeDtypeStruct((N, ow), jnp.float32),
  )(s0, s1, wt)


def _pad_edges(ei, ew):
  pad = EPAD - E
  row = jnp.concatenate([ei[0], jnp.zeros((pad,), jnp.int32)])
  col = jnp.concatenate([ei[1], jnp.zeros((pad,), jnp.int32)])
  w = jnp.concatenate([ew, jnp.zeros((pad,), jnp.float32)])
  return (row.reshape(EPAD // C, C), col.reshape(EPAD // C, C),
          w.reshape(EPAD // C, C))


def kernel(x, x_per, edge_weight, edge_weight_per, W0, W1, Wss,
           edge_index, edge_index_per):
  dk = jax.random.key(42)
  m0 = jnp.where(
      jax.random.bernoulli(jax.random.fold_in(dk, 0), 0.5, (N, D)), 2.0, 0.0)
  m1 = jnp.where(
      jax.random.bernoulli(jax.random.fold_in(dk, 1), 0.5, (N, D)), 2.0, 0.0)

  row, col, w2 = _pad_edges(edge_index, edge_weight)
  rowp, colp, wp2 = _pad_edges(edge_index_per, edge_weight_per)

  w0t = W0.T.reshape(2, 128, D)
  w1t = W1.T.reshape(2, 128, D)
  wsst = jnp.pad(Wss.T, ((0, 0), (0, 128 - Wss.shape[0]))).reshape(2, 128, 128)

  ea = (row, col, w2)
  eb = (rowp, colp, wp2)

  s0, s1, sp0, sp1 = _spmm2_sc(
      x[:, :128], x[:, 128:], x_per[:, :128], x_per[:, 128:], ea, eb)
  t0, t1 = _lin_relu_mask(s0, s1, w0t, m0)
  tp0, tp1 = _lin_relu_mask(sp0, sp1, w0t, m1)

  u0, u1, up0, up1 = _spmm2_sc(t0, t1, tp0, tp1, ea, eb)
  h = _lin_plain(u0, u1, w1t)
  hp = _lin_plain(up0, up1, wsst)[:, :Wss.shape[0]]

  return h, hp


# speedup vs baseline: 1.1549x; 1.1549x over previous
import functools

import jax
import jax.numpy as jnp
from jax import lax
from jax.experimental import pallas as pl
from jax.experimental.pallas import tpu as pltpu
from jax.experimental.pallas import tpu_sc as plsc

N = 10000
D = 256
E = 160000
C = 128
NSUB = 16
NCH_T = 80
EPAD = NSUB * NCH_T * C
ZR = 200
NZCH = N // ZR


def _spmm_sc(h_lo, h_hi, rows2, cols2, w2):
  mesh = plsc.VectorSubcoreMesh(core_axis_name="c", subcore_axis_name="s")

  @functools.partial(
      pl.kernel,
      out_type=[jax.ShapeDtypeStruct((N, 128), jnp.float32)] * 2,
      mesh=mesh,
      scratch_types=[
          pltpu.VMEM((NCH_T // 2, C), jnp.int32),
          pltpu.VMEM((NCH_T // 2, C), jnp.int32),
          pltpu.VMEM((NCH_T // 2, C), jnp.float32),
          pltpu.VMEM((C, 128), jnp.float32),
          pltpu.VMEM((C, 128), jnp.float32),
          pltpu.VMEM_SHARED((N, 128), jnp.float32),
          pltpu.SemaphoreType.DMA,
          pltpu.SemaphoreType.DMA,
      ],
  )
  def k(hlo, hhi, rws, cls, ws, olo, ohi,
        colr, rowr, wr, g0, g1, acc, gsem0, gsem1):
    c = lax.axis_index("c")
    s = lax.axis_index("s")
    hc = NCH_T // 2

    @pl.loop(0, 80)
    def _(i):
      for kk in range(8):
        g0[i, pl.ds(kk * 16, 16)] = jnp.zeros((16,), jnp.float32)

    zsrc = g0.at[pl.ds(0, 80)]

    @pl.loop(s, 125, step=NSUB)
    def _(j):
      pltpu.sync_copy(zsrc, acc.at[pl.ds(j * 80, 80)])

    plsc.subcore_barrier()

    def run(h_ref):
      bufs = ((g0, gsem0), (g1, gsem1))
      for half in range(2):
        tsl = pl.ds(s * NCH_T + half * hc, hc)
        pltpu.sync_copy(cls.at[tsl], colr)
        pltpu.sync_copy(rws.at[tsl], rowr)
        pltpu.sync_copy(ws.at[tsl], wr)

        pltpu.async_copy(h_ref.at[colr.at[0]], g0, gsem0)
        pltpu.async_copy(h_ref.at[colr.at[1]], g1, gsem1)

        @pl.loop(0, hc // 2)
        def _(i):
          for p, (gbuf, gsem) in enumerate(bufs):
            ck = 2 * i + p
            pltpu.make_async_copy(h_ref.at[colr.at[ck]], gbuf, gsem).wait()

            @pl.loop(0, C // 16)
            def _(b):
              wvec = wr[ck, pl.ds(b * 16, 16)]
              for ii in range(16):
                w = wvec[ii]
                for kk in range(8):
                  sl = pl.ds(kk * 16, 16)
                  gbuf[b * 16 + ii, sl] = gbuf[b * 16 + ii, sl] * w

            pltpu.sync_copy(gbuf, acc.at[rowr.at[ck]], add=True)

            @pl.when(ck + 2 < hc)
            def _():
              pltpu.async_copy(h_ref.at[colr.at[ck + 2]], gbuf, gsem)

    @pl.when(c == 0)
    def _():
      run(hlo)

    @pl.when(c == 1)
    def _():
      run(hhi)

    plsc.subcore_barrier()

    @pl.loop(s, NZCH, step=NSUB)
    def _(j):
      sl = pl.ds(j * ZR, ZR)

      @pl.when(c == 0)
      def _():
        pltpu.sync_copy(acc.at[sl], olo.at[sl])

      @pl.when(c == 1)
      def _():
        pltpu.sync_copy(acc.at[sl], ohi.at[sl])

  return k(h_lo, h_hi, rows2, cols2, w2)


_BM = 1000


def _lin_relu_mask(s0, s1, wt, mask):
  def body(a0, a1, w, m, o0, o1):
    acc = jnp.dot(a0[...], w[0], preferred_element_type=jnp.float32)
    acc = acc + jnp.dot(a1[...], w[1], preferred_element_type=jnp.float32)
    y = jnp.maximum(acc, 0.0) * m[...]
    o0[...] = y[:, :128]
    o1[...] = y[:, 128:]

  return pl.pallas_call(
      body,
      grid=(N // _BM,),
      in_specs=[
          pl.BlockSpec((_BM, 128), lambda i: (i, 0)),
          pl.BlockSpec((_BM, 128), lambda i: (i, 0)),
          pl.BlockSpec((2, 128, D), lambda i: (0, 0, 0)),
          pl.BlockSpec((_BM, D), lambda i: (i, 0)),
      ],
      out_specs=[
          pl.BlockSpec((_BM, 128), lambda i: (i, 0)),
          pl.BlockSpec((_BM, 128), lambda i: (i, 0)),
      ],
      out_shape=[jax.ShapeDtypeStruct((N, 128), jnp.float32)] * 2,
  )(s0, s1, wt, mask)


def _lin_plain(s0, s1, wt):
  ow = wt.shape[2]

  def body(a0, a1, w, o):
    acc = jnp.dot(a0[...], w[0], preferred_element_type=jnp.float32)
    acc = acc + jnp.dot(a1[...], w[1], preferred_element_type=jnp.float32)
    o[...] = acc

  return pl.pallas_call(
      body,
      grid=(N // _BM,),
      in_specs=[
          pl.BlockSpec((_BM, 128), lambda i: (i, 0)),
          pl.BlockSpec((_BM, 128), lambda i: (i, 0)),
          pl.BlockSpec((2, 128, ow), lambda i: (0, 0, 0)),
      ],
      out_specs=pl.BlockSpec((_BM, ow), lambda i: (i, 0)),
      out_shape=jax.ShapeDtypeStruct((N, ow), jnp.float32),
  )(s0, s1, wt)


def _pad_edges(ei, ew):
  pad = EPAD - E
  row = jnp.concatenate([ei[0], jnp.zeros((pad,), jnp.int32)])
  col = jnp.concatenate([ei[1], jnp.zeros((pad,), jnp.int32)])
  w = jnp.concatenate([ew, jnp.zeros((pad,), jnp.float32)])
  return (row.reshape(EPAD // C, C), col.reshape(EPAD // C, C),
          w.reshape(EPAD // C, C))


def kernel(x, x_per, edge_weight, edge_weight_per, W0, W1, Wss,
           edge_index, edge_index_per):
  dk = jax.random.key(42)
  m0 = jnp.where(
      jax.random.bernoulli(jax.random.fold_in(dk, 0), 0.5, (N, D)), 2.0, 0.0)
  m1 = jnp.where(
      jax.random.bernoulli(jax.random.fold_in(dk, 1), 0.5, (N, D)), 2.0, 0.0)

  row, col, w2 = _pad_edges(edge_index, edge_weight)
  rowp, colp, wp2 = _pad_edges(edge_index_per, edge_weight_per)

  w0t = W0.T.reshape(2, 128, D)
  w1t = W1.T.reshape(2, 128, D)
  wsst = jnp.pad(Wss.T, ((0, 0), (0, 128 - Wss.shape[0]))).reshape(2, 128, 128)

  s0, s1 = _spmm_sc(x[:, :128], x[:, 128:], row, col, w2)
  t0, t1 = _lin_relu_mask(s0, s1, w0t, m0)
  u0, u1 = _spmm_sc(t0, t1, row, col, w2)
  h = _lin_plain(u0, u1, w1t)

  sp0, sp1 = _spmm_sc(x_per[:, :128], x_per[:, 128:], rowp, colp, wp2)
  tp0, tp1 = _lin_relu_mask(sp0, sp1, w0t, m1)
  up0, up1 = _spmm_sc(tp0, tp1, rowp, colp, wp2)
  hp = _lin_plain(up0, up1, wsst)[:, :Wss.shape[0]]

  return h, hp
